# baseline (speedup 1.0000x reference)
import functools

import jax
import jax.numpy as jnp
from jax import lax
from jax.experimental import pallas as pl
from jax.experimental.pallas import tpu as pltpu
from jax.experimental.pallas import tpu_sc as plsc


def _sc_gather(table, idx, B, V, D):
    info = plsc.get_sparse_core_info()
    NW = info.num_cores * info.num_subcores
    b_per_w = B // NW
    mesh = plsc.VectorSubcoreMesh(core_axis_name="c", subcore_axis_name="s")

    @functools.partial(
        pl.kernel,
        mesh=mesh,
        out_type=jax.ShapeDtypeStruct((B, D), jnp.float32),
        scratch_types=[
            pltpu.VMEM((b_per_w,), jnp.int32),
            pltpu.VMEM((b_per_w, D), jnp.float32),
            pltpu.SemaphoreType.DMA,
        ],
        compiler_params=pltpu.CompilerParams(use_tc_tiling_on_sc=False),
    )
    def gather_kernel(table_hbm, idx_hbm, out_hbm, idx_v, rows_v, sem):
        wid = lax.axis_index("s") * info.num_cores + lax.axis_index("c")
        base = wid * b_per_w
        pltpu.sync_copy(idx_hbm.at[pl.ds(base, b_per_w)], idx_v)
        pltpu.async_copy(table_hbm.at[idx_v], rows_v, sem).wait()
        pltpu.sync_copy(rows_v, out_hbm.at[pl.ds(base, b_per_w)])

    return gather_kernel(table, idx)


def _tc_project(e_aug, AT, B, V, DA, TN):
    def proj_kernel(a_ref, e_ref, o_ref):
        o_ref[...] = lax.dot_general(
            a_ref[...], e_ref[...],
            dimension_numbers=(((0,), (1,)), ((), ())),
            preferred_element_type=jnp.float32,
        )

    return pl.pallas_call(
        proj_kernel,
        grid=(pl.cdiv(V, TN),),
        in_specs=[
            pl.BlockSpec((DA, TN), lambda i: (0, i)),
            pl.BlockSpec((B, DA), lambda i: (0, 0)),
        ],
        out_specs=pl.BlockSpec((TN, B), lambda i: (i, 0)),
        out_shape=jax.ShapeDtypeStruct((V, B), jnp.float32),
        compiler_params=pltpu.CompilerParams(
            dimension_semantics=("parallel",),
            vmem_limit_bytes=100 * 1024 * 1024,
        ),
    )(AT, e_aug)


def kernel(target, emb_table, W, b):
    V, D = emb_table.shape
    B = target.shape[0]
    idx = target.astype(jnp.int32)
    embed = _sc_gather(emb_table, idx, B, V, D)
    AT = jnp.concatenate([W.T, b[None, :]], axis=0)
    e_aug = jnp.concatenate([embed, jnp.ones((B, 1), jnp.float32)], axis=1)
    out_t = _tc_project(e_aug, AT, B, V, D + 1, TN=4096)
    return out_t.T


# speedup vs baseline: 1.0415x; 1.0415x over previous
import functools

import jax
import jax.numpy as jnp
from jax import lax
from jax.experimental import pallas as pl
from jax.experimental.pallas import tpu as pltpu
from jax.experimental.pallas import tpu_sc as plsc


def _sc_gather(table, idx, B, V, D):
    info = plsc.get_sparse_core_info()
    NW = info.num_cores * info.num_subcores
    b_per_w = B // NW
    mesh = plsc.VectorSubcoreMesh(core_axis_name="c", subcore_axis_name="s")

    @functools.partial(
        pl.kernel,
        mesh=mesh,
        out_type=jax.ShapeDtypeStruct((B, D), jnp.float32),
        scratch_types=[
            pltpu.VMEM((b_per_w,), jnp.int32),
            pltpu.VMEM((b_per_w, D), jnp.float32),
            pltpu.SemaphoreType.DMA,
        ],
        compiler_params=pltpu.CompilerParams(use_tc_tiling_on_sc=False),
    )
    def gather_kernel(table_hbm, idx_hbm, out_hbm, idx_v, rows_v, sem):
        wid = lax.axis_index("s") * info.num_cores + lax.axis_index("c")
        base = wid * b_per_w
        pltpu.sync_copy(idx_hbm.at[pl.ds(base, b_per_w)], idx_v)
        pltpu.async_copy(table_hbm.at[idx_v], rows_v, sem).wait()
        pltpu.sync_copy(rows_v, out_hbm.at[pl.ds(base, b_per_w)])

    return gather_kernel(table, idx)


def _tc_project(e_aug, AT, B, V, DA, TN):
    def proj_kernel(a_ref, e_ref, o_ref):
        o_ref[...] = lax.dot_general(
            a_ref[...], e_ref[...],
            dimension_numbers=(((0,), (1,)), ((), ())),
            preferred_element_type=jnp.float32,
        )

    return pl.pallas_call(
        proj_kernel,
        grid=(pl.cdiv(V, TN),),
        in_specs=[
            pl.BlockSpec((DA, TN), lambda i: (0, i)),
            pl.BlockSpec((B, DA), lambda i: (0, 0)),
        ],
        out_specs=pl.BlockSpec((TN, B), lambda i: (i, 0)),
        out_shape=jax.ShapeDtypeStruct((V, B), jnp.float32),
        compiler_params=pltpu.CompilerParams(
            dimension_semantics=("parallel",),
            allow_input_fusion=(True, False),
            vmem_limit_bytes=100 * 1024 * 1024,
        ),
    )(AT, e_aug)


def kernel(target, emb_table, W, b):
    V, D = emb_table.shape
    B = target.shape[0]
    idx = target.astype(jnp.int32)
    embed = _sc_gather(emb_table, idx, B, V, D)
    AT = jnp.concatenate([W.T, b[None, :]], axis=0)
    e_aug = jnp.concatenate([embed, jnp.ones((B, 1), jnp.float32)], axis=1)
    out_t = _tc_project(e_aug, AT, B, V, D + 1, TN=2048)
    return out_t.T


# speedup vs baseline: 1.0429x; 1.0014x over previous
import functools

import jax
import jax.numpy as jnp
from jax import lax
from jax.experimental import pallas as pl
from jax.experimental.pallas import tpu as pltpu
from jax.experimental.pallas import tpu_sc as plsc


def _sc_gather(table, idx, B, V, D):
    info = plsc.get_sparse_core_info()
    NW = info.num_cores * info.num_subcores
    b_per_w = B // NW
    mesh = plsc.VectorSubcoreMesh(core_axis_name="c", subcore_axis_name="s")

    @functools.partial(
        pl.kernel,
        mesh=mesh,
        out_type=jax.ShapeDtypeStruct((B, D), jnp.float32),
        scratch_types=[
            pltpu.VMEM((b_per_w,), jnp.int32),
            pltpu.VMEM((b_per_w, D), jnp.float32),
            pltpu.SemaphoreType.DMA,
        ],
        compiler_params=pltpu.CompilerParams(use_tc_tiling_on_sc=False),
    )
    def gather_kernel(table_hbm, idx_hbm, out_hbm, idx_v, rows_v, sem):
        wid = lax.axis_index("s") * info.num_cores + lax.axis_index("c")
        base = wid * b_per_w
        pltpu.sync_copy(idx_hbm.at[pl.ds(base, b_per_w)], idx_v)
        pltpu.async_copy(table_hbm.at[idx_v], rows_v, sem).wait()
        pltpu.sync_copy(rows_v, out_hbm.at[pl.ds(base, b_per_w)])

    return gather_kernel(table, idx)


def _tc_project(e_aug, AT, B, V, DA, TN):
    def proj_kernel(a_ref, e_ref, o_ref):
        o_ref[...] = lax.dot_general(
            a_ref[...], e_ref[...],
            dimension_numbers=(((0,), (1,)), ((), ())),
            preferred_element_type=jnp.float32,
        )

    return pl.pallas_call(
        proj_kernel,
        grid=(pl.cdiv(V, TN),),
        in_specs=[
            pl.BlockSpec((DA, TN), lambda i: (0, i)),
            pl.BlockSpec((B, DA), lambda i: (0, 0)),
        ],
        out_specs=pl.BlockSpec((TN, B), lambda i: (i, 0)),
        out_shape=jax.ShapeDtypeStruct((V, B), jnp.float32),
        compiler_params=pltpu.CompilerParams(
            dimension_semantics=("parallel",),
            allow_input_fusion=(True, False),
            vmem_limit_bytes=100 * 1024 * 1024,
            skip_device_barrier=True,
        ),
    )(AT, e_aug)


def kernel(target, emb_table, W, b):
    V, D = emb_table.shape
    B = target.shape[0]
    idx = target.astype(jnp.int32)
    embed = _sc_gather(emb_table, idx, B, V, D)
    AT = jnp.concatenate([W.T, b[None, :]], axis=0)
    e_aug = jnp.concatenate([embed, jnp.ones((B, 1), jnp.float32)], axis=1)
    out_t = _tc_project(e_aug, AT, B, V, D + 1, TN=2048)
    return out_t.T


# speedup vs baseline: 1.0435x; 1.0006x over previous
import functools

import jax
import jax.numpy as jnp
from jax import lax
from jax.experimental import pallas as pl
from jax.experimental.pallas import tpu as pltpu
from jax.experimental.pallas import tpu_sc as plsc


def _sc_gather(table, idx, B, V, D):
    info = plsc.get_sparse_core_info()
    NW = info.num_cores * info.num_subcores
    b_per_w = B // NW
    mesh = plsc.VectorSubcoreMesh(core_axis_name="c", subcore_axis_name="s")

    @functools.partial(
        pl.kernel,
        mesh=mesh,
        out_type=jax.ShapeDtypeStruct((B, D), jnp.float32),
        scratch_types=[
            pltpu.VMEM((b_per_w,), jnp.int32),
            pltpu.VMEM((b_per_w, D), jnp.float32),
            pltpu.SemaphoreType.DMA,
        ],
        compiler_params=pltpu.CompilerParams(use_tc_tiling_on_sc=False, skip_device_barrier=True),
    )
    def gather_kernel(table_hbm, idx_hbm, out_hbm, idx_v, rows_v, sem):
        wid = lax.axis_index("s") * info.num_cores + lax.axis_index("c")
        base = wid * b_per_w
        pltpu.sync_copy(idx_hbm.at[pl.ds(base, b_per_w)], idx_v)
        pltpu.async_copy(table_hbm.at[idx_v], rows_v, sem).wait()
        pltpu.sync_copy(rows_v, out_hbm.at[pl.ds(base, b_per_w)])

    return gather_kernel(table, idx)


def _tc_project(e_aug, AT, B, V, DA, TN):
    def proj_kernel(a_ref, e_ref, o_ref):
        o_ref[...] = lax.dot_general(
            a_ref[...], e_ref[...],
            dimension_numbers=(((0,), (1,)), ((), ())),
            preferred_element_type=jnp.float32,
        )

    return pl.pallas_call(
        proj_kernel,
        grid=(pl.cdiv(V, TN),),
        in_specs=[
            pl.BlockSpec((DA, TN), lambda i: (0, i)),
            pl.BlockSpec((B, DA), lambda i: (0, 0)),
        ],
        out_specs=pl.BlockSpec((TN, B), lambda i: (i, 0)),
        out_shape=jax.ShapeDtypeStruct((V, B), jnp.float32),
        compiler_params=pltpu.CompilerParams(
            dimension_semantics=("parallel",),
            allow_input_fusion=(True, False),
            vmem_limit_bytes=100 * 1024 * 1024,
            skip_device_barrier=True,
        ),
    )(AT, e_aug)


def kernel(target, emb_table, W, b):
    V, D = emb_table.shape
    B = target.shape[0]
    idx = target.astype(jnp.int32)
    embed = _sc_gather(emb_table, idx, B, V, D)
    AT = jnp.concatenate([W.T, b[None, :]], axis=0)
    e_aug = jnp.concatenate([embed, jnp.ones((B, 1), jnp.float32)], axis=1)
    out_t = _tc_project(e_aug, AT, B, V, D + 1, TN=2048)
    return out_t.T
